# baseline (speedup 1.0000x reference)
import functools

import jax
import jax.numpy as jnp
from jax import lax
from jax.experimental import pallas as pl
from jax.experimental.pallas import tpu as pltpu
from jax.experimental.pallas import tpu_sc as plsc

_NC, _NS = 2, 16
_GP = 9
_TC_BLK = 12


def _sc_body(y_hbm, out_hbm, buf, avg, s_in, s_out, s_avg):
    wid = lax.axis_index("s") * _NC + lax.axis_index("c")
    r0 = wid * 8
    pltpu.async_copy(
        y_hbm.at[pl.ds(3, _GP), pl.ds(r0, 8), :], buf, s_in).wait()
    store = pltpu.async_copy(
        buf, out_hbm.at[pl.ds(3, _GP), pl.ds(r0, 8), :], s_out)

    def _mean_step(t, carry):
        for u in range(4):
            s = t * 4 + u
            r = s // 32
            l = (s % 32) * 16
            for c in range(3):
                v = (buf[c, r, pl.ds(l, 16)]
                     + buf[c + 3, r, pl.ds(l, 16)]
                     + buf[c + 6, r, pl.ds(l, 16)]) * (1.0 / 3.0)
                avg[c, r, pl.ds(l, 16)] = v
        return carry

    lax.fori_loop(0, 8 * 32 // 4, _mean_step, 0)
    pltpu.async_copy(
        avg, out_hbm.at[pl.ds(0, 3), pl.ds(r0, 8), :], s_avg).wait()
    store.wait()


def _make_sc_call(planes, B, F):
    mesh = plsc.VectorSubcoreMesh(core_axis_name="c", subcore_axis_name="s")
    return functools.partial(
        pl.kernel,
        out_type=jax.ShapeDtypeStruct((planes, B, F), jnp.float32),
        mesh=mesh,
        scratch_types=[
            pltpu.VMEM((_GP, 8, F), jnp.float32),
            pltpu.VMEM((3, 8, F), jnp.float32),
            pltpu.SemaphoreType.DMA,
            pltpu.SemaphoreType.DMA,
            pltpu.SemaphoreType.DMA,
        ],
    )(_sc_body)


def _tc_body(x_ref, alias_ref, o_ref):
    del alias_ref
    o_ref[...] = x_ref[...]


def kernel(x):
    B, F, J, C = x.shape
    planes = J * C
    y = jnp.transpose(x, (2, 3, 0, 1)).reshape(planes, B, F)
    out0 = _make_sc_call(planes, B, F)(y)
    out = pl.pallas_call(
        _tc_body,
        grid=((planes - 12) // _TC_BLK,),
        in_specs=[
            pl.BlockSpec((_TC_BLK, B, F), lambda i: (i + 1, 0, 0)),
            pl.BlockSpec(memory_space=pl.ANY),
        ],
        out_specs=pl.BlockSpec((_TC_BLK, B, F), lambda i: (i + 1, 0, 0)),
        out_shape=jax.ShapeDtypeStruct((planes, B, F), x.dtype),
        input_output_aliases={1: 0},
    )(y, out0)
    return jnp.transpose(out.reshape(J, C, B, F), (2, 3, 0, 1))


# speedup vs baseline: 1.0618x; 1.0618x over previous
import functools

import jax
import jax.numpy as jnp
from jax import lax
from jax.experimental import pallas as pl
from jax.experimental.pallas import tpu as pltpu
from jax.experimental.pallas import tpu_sc as plsc

_NC, _NS = 2, 16
_GP = 9
_TC_BLK = 12


def _sc_body(y_hbm, out_hbm, buf, avg, s_in, s_out, s_avg):
    wid = lax.axis_index("s") * _NC + lax.axis_index("c")
    r0 = wid * 8
    pltpu.async_copy(
        y_hbm.at[pl.ds(3, _GP), pl.ds(r0, 8), :], buf, s_in).wait()
    store = pltpu.async_copy(
        buf, out_hbm.at[pl.ds(3, _GP), pl.ds(r0, 8), :], s_out)

    def _mean_step(t, carry):
        r = t // 32
        l = (t % 32) * 16
        for c in range(3):
            v = (buf[c, r, pl.ds(l, 16)]
                 + buf[c + 3, r, pl.ds(l, 16)]
                 + buf[c + 6, r, pl.ds(l, 16)]) * (1.0 / 3.0)
            avg[c, r, pl.ds(l, 16)] = v
        return carry

    lax.fori_loop(0, 8 * 32, _mean_step, 0)
    pltpu.async_copy(
        avg, out_hbm.at[pl.ds(0, 3), pl.ds(r0, 8), :], s_avg).wait()
    store.wait()


def _make_sc_call(planes, B, F):
    mesh = plsc.VectorSubcoreMesh(core_axis_name="c", subcore_axis_name="s")
    return functools.partial(
        pl.kernel,
        out_type=jax.ShapeDtypeStruct((planes, B, F), jnp.float32),
        mesh=mesh,
        scratch_types=[
            pltpu.VMEM((_GP, 8, F), jnp.float32),
            pltpu.VMEM((3, 8, F), jnp.float32),
            pltpu.SemaphoreType.DMA,
            pltpu.SemaphoreType.DMA,
            pltpu.SemaphoreType.DMA,
        ],
    )(_sc_body)


def _tc_body(x_ref, alias_ref, o_ref):
    del alias_ref
    o_ref[...] = x_ref[...]


def kernel(x):
    B, F, J, C = x.shape
    planes = J * C
    y = jnp.transpose(x, (2, 3, 0, 1)).reshape(planes, B, F)
    out0 = _make_sc_call(planes, B, F)(y)
    out = pl.pallas_call(
        _tc_body,
        grid=((planes - 12) // _TC_BLK,),
        in_specs=[
            pl.BlockSpec((_TC_BLK, B, F), lambda i: (i + 1, 0, 0)),
            pl.BlockSpec(memory_space=pl.ANY),
        ],
        out_specs=pl.BlockSpec((_TC_BLK, B, F), lambda i: (i + 1, 0, 0)),
        out_shape=jax.ShapeDtypeStruct((planes, B, F), x.dtype),
        input_output_aliases={1: 0},
    )(y, out0)
    return jnp.transpose(out.reshape(J, C, B, F), (2, 3, 0, 1))
